# baseline (speedup 1.0000x reference)
import functools

import jax
import jax.numpy as jnp
from jax import lax
from jax.experimental import pallas as pl
from jax.experimental.pallas import tpu as pltpu
from jax.experimental.pallas import tpu_sc as plsc

N = 16384
NEWS_DIM = 64
CAT_DIM = 16
D = NEWS_DIM + CAT_DIM
CH = 128


@functools.lru_cache(maxsize=1)
def _build_sc_kernel():
    info = plsc.get_sparse_core_info()
    nc, ns = info.num_cores, info.num_subcores
    nw = nc * ns
    bpw = N // nw
    n_chunks = bpw // CH
    mesh = plsc.VectorSubcoreMesh(core_axis_name="c", subcore_axis_name="s")

    @functools.partial(
        pl.kernel,
        mesh=mesh,
        out_type=jax.ShapeDtypeStruct((N, D), jnp.float32),
        compiler_params=pltpu.CompilerParams(use_tc_tiling_on_sc=False),
        scratch_types=[
            pltpu.VMEM((n_chunks, CH), jnp.int32),
            pltpu.VMEM((n_chunks, CH), jnp.int32),
            pltpu.VMEM((bpw,), jnp.float32),
            pltpu.VMEM((bpw, NEWS_DIM), jnp.float32),
            pltpu.VMEM((bpw, CAT_DIM), jnp.float32),
            pltpu.VMEM((bpw, D), jnp.float32),
            pltpu.SemaphoreType.DMA,
        ],
    )
    def sc_kernel(news_ids_hbm, cat_ids_hbm, mask_hbm, news_tab_hbm,
                  cat_tab_hbm, out_hbm,
                  nidx_v, cidx_v, mask_v, news_v, cat_v, out_v, sem):
        wid = lax.axis_index("s") * nc + lax.axis_index("c")
        base = wid * bpw
        for j in range(n_chunks):
            pltpu.sync_copy(news_ids_hbm.at[pl.ds(base + j * CH, CH)],
                            nidx_v.at[j])
            pltpu.sync_copy(cat_ids_hbm.at[pl.ds(base + j * CH, CH)],
                            cidx_v.at[j])
        pltpu.sync_copy(mask_hbm.at[pl.ds(base, bpw)], mask_v)

        copies = []
        for j in range(n_chunks):
            copies.append(pltpu.async_copy(
                news_tab_hbm.at[nidx_v.at[j]],
                news_v.at[pl.ds(j * CH, CH)], sem))
            copies.append(pltpu.async_copy(
                cat_tab_hbm.at[cidx_v.at[j]],
                cat_v.at[pl.ds(j * CH, CH)], sem))
        for c in copies:
            c.wait()

        def body(g, carry):
            mv = mask_v[pl.ds(g * 16, 16)]
            for k in range(16):
                i = g * 16 + k
                m = mv[k]
                for j in range(NEWS_DIM // 16):
                    out_v[i, pl.ds(j * 16, 16)] = (
                        news_v[i, pl.ds(j * 16, 16)] * m)
                out_v[i, pl.ds(NEWS_DIM, CAT_DIM)] = (
                    cat_v[i, pl.ds(0, CAT_DIM)] * m)
            return carry
        lax.fori_loop(0, bpw // 16, body, 0)

        pltpu.sync_copy(out_v, out_hbm.at[pl.ds(base, bpw)])

    return sc_kernel


def kernel(news_ids, category_ids, delta_t, mask, news_table, category_table):
    sc = _build_sc_kernel()
    X = sc(news_ids, category_ids, mask, news_table, category_table)
    return (X, delta_t)
